# baseline (speedup 1.0000x reference)
import functools

import jax
import jax.numpy as jnp
from jax import lax
from jax.experimental import pallas as pl
from jax.experimental.pallas import tpu as pltpu
from jax.experimental.pallas import tpu_sc as plsc

EMB_DIM = 64
BATCH = 4096
SEQ = 200
SEQ_PAD = 208

_info = plsc.get_sparse_core_info()
NC, NS, NL = _info.num_cores, _info.num_subcores, _info.num_lanes
NW = NC * NS
BPW = BATCH // NW


HSEQ = SEQ_PAD // 2


def _sc_pool_body(ids_hbm, emb_hbm, sums_hbm,
                  idxa0, idxa1, idxb0, idxb1,
                  rowsa0, rowsa1, rowsb0, rowsb1, sums_v,
                  sia0, sia1, sib0, sib1, sga0, sga1, sgb0, sgb1):
    wid = lax.axis_index("s") * NC + lax.axis_index("c")
    base = wid * BPW

    bufs_a = (idxa0, idxa1, rowsa0, rowsa1, sia0, sia1, sga0, sga1)
    bufs_b = (idxb0, idxb1, rowsb0, rowsb1, sib0, sib1, sgb0, sgb1)

    def ids_copies(b, bufs):
        idx0, idx1, _, _, si0, si1, _, _ = bufs
        off = pl.multiple_of((base + b) * SEQ_PAD, 8)
        return (pltpu.make_async_copy(ids_hbm.at[pl.ds(off, HSEQ)], idx0, si0),
                pltpu.make_async_copy(
                    ids_hbm.at[pl.ds(off + HSEQ, HSEQ)], idx1, si1))

    def gather_copies(bufs):
        idx0, idx1, rows0, rows1, _, _, sg0, sg1 = bufs
        return (pltpu.make_async_copy(emb_hbm.at[idx0], rows0, sg0),
                pltpu.make_async_copy(emb_hbm.at[idx1], rows1, sg1))

    def accumulate(b, bufs):
        _, _, rows0, rows1, _, _, _, _ = bufs

        def row_body(r, accs):
            a0, a1, a2, a3 = accs
            return (a0 + rows0[r, pl.ds(0, 16)] + rows1[r, pl.ds(0, 16)],
                    a1 + rows0[r, pl.ds(16, 16)] + rows1[r, pl.ds(16, 16)],
                    a2 + rows0[r, pl.ds(32, 16)] + rows1[r, pl.ds(32, 16)],
                    a3 + rows0[r, pl.ds(48, 16)] + rows1[r, pl.ds(48, 16)])

        z = jnp.zeros((16,), jnp.float32)
        a0, a1, a2, a3 = lax.fori_loop(0, HSEQ, row_body, (z, z, z, z),
                                       unroll=4)
        sums_v[b, pl.ds(0, 16)] = a0
        sums_v[b, pl.ds(16, 16)] = a1
        sums_v[b, pl.ds(32, 16)] = a2
        sums_v[b, pl.ds(48, 16)] = a3

    for c in ids_copies(0, bufs_a):
        c.start()
    for c in ids_copies(1, bufs_b):
        c.start()
    for c in ids_copies(0, bufs_a):
        c.wait()
    for c in gather_copies(bufs_a):
        c.start()

    def pair_body(g, carry):
        b0 = 2 * g
        not_last = g < BPW // 2 - 1

        for c in ids_copies(b0 + 1, bufs_b):
            c.wait()
        for c in gather_copies(bufs_b):
            c.start()
        for c in gather_copies(bufs_a):
            c.wait()

        @pl.when(not_last)
        def _():
            for c in ids_copies(b0 + 2, bufs_a):
                c.start()

        accumulate(b0, bufs_a)

        @pl.when(not_last)
        def _():
            for c in ids_copies(b0 + 2, bufs_a):
                c.wait()
            for c in gather_copies(bufs_a):
                c.start()

        for c in gather_copies(bufs_b):
            c.wait()

        @pl.when(not_last)
        def _():
            for c in ids_copies(b0 + 3, bufs_b):
                c.start()

        accumulate(b0 + 1, bufs_b)
        return carry

    lax.fori_loop(0, BPW // 2, pair_body, 0)
    pltpu.sync_copy(sums_v, sums_hbm.at[pl.ds(base, BPW)])


_sc_pool = functools.partial(
    pl.kernel,
    out_type=jax.ShapeDtypeStruct((BATCH, EMB_DIM), jnp.float32),
    mesh=plsc.VectorSubcoreMesh(core_axis_name="c", subcore_axis_name="s"),
    compiler_params=pltpu.CompilerParams(use_tc_tiling_on_sc=False),
    scratch_types=[
        pltpu.VMEM((HSEQ,), jnp.int32),
        pltpu.VMEM((HSEQ,), jnp.int32),
        pltpu.VMEM((HSEQ,), jnp.int32),
        pltpu.VMEM((HSEQ,), jnp.int32),
        pltpu.VMEM((HSEQ, EMB_DIM), jnp.float32),
        pltpu.VMEM((HSEQ, EMB_DIM), jnp.float32),
        pltpu.VMEM((HSEQ, EMB_DIM), jnp.float32),
        pltpu.VMEM((HSEQ, EMB_DIM), jnp.float32),
        pltpu.VMEM((BPW, EMB_DIM), jnp.float32),
        pltpu.SemaphoreType.DMA,
        pltpu.SemaphoreType.DMA,
        pltpu.SemaphoreType.DMA,
        pltpu.SemaphoreType.DMA,
        pltpu.SemaphoreType.DMA,
        pltpu.SemaphoreType.DMA,
        pltpu.SemaphoreType.DMA,
        pltpu.SemaphoreType.DMA,
    ],
)(_sc_pool_body)


def _mlp_body(ids_ref, sums_ref, w1_ref, b1_ref, w2_ref, b2_ref, out_ref):
    cnt = jnp.sum((ids_ref[...] != 0).astype(jnp.float32), axis=1,
                  keepdims=True)
    avg = sums_ref[...] / jnp.maximum(cnt, 1e-6)
    h = jnp.dot(avg, w1_ref[...], preferred_element_type=jnp.float32,
                precision=lax.Precision.HIGHEST) + b1_ref[...]
    h = jnp.maximum(h, 0.0)
    out_ref[...] = jnp.dot(h, w2_ref[...], preferred_element_type=jnp.float32,
                           precision=lax.Precision.HIGHEST) + b2_ref[...]


def kernel(ids, emb, W1, b1, W2, b2):
    ids = ids.astype(jnp.int32)
    idsp = jnp.pad(ids, ((0, 0), (0, SEQ_PAD - SEQ)))
    sums = _sc_pool(idsp.reshape(-1), emb)

    blk = 512
    grid = (BATCH // blk,)
    hidden = W1.shape[1]
    out_dim = W2.shape[1]
    out = pl.pallas_call(
        _mlp_body,
        grid=grid,
        in_specs=[
            pl.BlockSpec((blk, SEQ_PAD), lambda i: (i, 0)),
            pl.BlockSpec((blk, EMB_DIM), lambda i: (i, 0)),
            pl.BlockSpec((EMB_DIM, hidden), lambda i: (0, 0)),
            pl.BlockSpec((1, hidden), lambda i: (0, 0)),
            pl.BlockSpec((hidden, out_dim), lambda i: (0, 0)),
            pl.BlockSpec((1, out_dim), lambda i: (0, 0)),
        ],
        out_specs=pl.BlockSpec((blk, out_dim), lambda i: (i, 0)),
        out_shape=jax.ShapeDtypeStruct((BATCH, out_dim), jnp.float32),
    )(idsp, sums, W1, b1[None, :], W2, b2[None, :])
    return out


# speedup vs baseline: 1.9068x; 1.9068x over previous
import functools

import jax
import jax.numpy as jnp
from jax import lax
from jax.experimental import pallas as pl
from jax.experimental.pallas import tpu as pltpu
from jax.experimental.pallas import tpu_sc as plsc

EMB_DIM = 64
BATCH = 4096
SEQ = 200
SA = 128
SB = SEQ - SA

_info = plsc.get_sparse_core_info()
NC, NS, NL = _info.num_cores, _info.num_subcores, _info.num_lanes
NW = NC * NS
BPW = BATCH // NW

G = 4
NGRP = BPW // G
NPAIR = NGRP // 2


def _sc_pool_body(ids_hbm, emb_hbm, sums_hbm,
                  idx0, idx1, ra0, rb0, ra1, rb1, sums_v,
                  semi0, semi1, semg0, semg1):
    wid = lax.axis_index("s") * NC + lax.axis_index("c")
    base = wid * BPW

    def idx_copy(g, idx, semi):
        return pltpu.make_async_copy(
            ids_hbm.at[pl.ds(base + g * G, G)], idx, semi)

    def gather_copies(idx, ra, rb, semg):
        cps = []
        for b in range(G):
            cps.append(pltpu.make_async_copy(
                emb_hbm.at[idx.at[b, pl.ds(0, SA)]], ra.at[b], semg))
            cps.append(pltpu.make_async_copy(
                emb_hbm.at[idx.at[b, pl.ds(SA, SB)]], rb.at[b], semg))
        return cps

    def accumulate(g, ra, rb):
        for b in range(G):
            def body_a(r, accs):
                return (accs[0] + ra[b, r, pl.ds(0, 16)],
                        accs[1] + ra[b, r, pl.ds(16, 16)],
                        accs[2] + ra[b, r, pl.ds(32, 16)],
                        accs[3] + ra[b, r, pl.ds(48, 16)])

            def body_b(r, accs):
                return (accs[0] + rb[b, r, pl.ds(0, 16)],
                        accs[1] + rb[b, r, pl.ds(16, 16)],
                        accs[2] + rb[b, r, pl.ds(32, 16)],
                        accs[3] + rb[b, r, pl.ds(48, 16)])

            z = jnp.zeros((16,), jnp.float32)
            a = lax.fori_loop(0, SA, body_a, (z, z, z, z), unroll=8)
            a = lax.fori_loop(0, SB, body_b, a, unroll=8)
            slot = g * G + b
            sums_v[slot, pl.ds(0, 16)] = a[0]
            sums_v[slot, pl.ds(16, 16)] = a[1]
            sums_v[slot, pl.ds(32, 16)] = a[2]
            sums_v[slot, pl.ds(48, 16)] = a[3]

    idx_copy(0, idx0, semi0).start()
    idx_copy(1, idx1, semi1).start()
    idx_copy(0, idx0, semi0).wait()
    for c in gather_copies(idx0, ra0, rb0, semg0):
        c.start()

    def pair_body(i, carry):
        g0 = 2 * i

        idx_copy(g0 + 1, idx1, semi1).wait()
        for c in gather_copies(idx1, ra1, rb1, semg1):
            c.start()
        for c in gather_copies(idx0, ra0, rb0, semg0):
            c.wait()

        @pl.when(g0 + 2 < NGRP)
        def _():
            idx_copy(g0 + 2, idx0, semi0).start()

        accumulate(g0, ra0, rb0)

        @pl.when(g0 + 2 < NGRP)
        def _():
            idx_copy(g0 + 2, idx0, semi0).wait()
            for c in gather_copies(idx0, ra0, rb0, semg0):
                c.start()

        for c in gather_copies(idx1, ra1, rb1, semg1):
            c.wait()

        @pl.when(g0 + 3 < NGRP)
        def _():
            idx_copy(g0 + 3, idx1, semi1).start()

        accumulate(g0 + 1, ra1, rb1)
        return carry

    lax.fori_loop(0, NPAIR, pair_body, 0)
    pltpu.sync_copy(sums_v, sums_hbm.at[pl.ds(base, BPW)])


_sc_pool = functools.partial(
    pl.kernel,
    out_type=jax.ShapeDtypeStruct((BATCH, EMB_DIM), jnp.float32),
    mesh=plsc.VectorSubcoreMesh(core_axis_name="c", subcore_axis_name="s"),
    compiler_params=pltpu.CompilerParams(use_tc_tiling_on_sc=False),
    scratch_types=[
        pltpu.VMEM((G, SEQ), jnp.int32),
        pltpu.VMEM((G, SEQ), jnp.int32),
        pltpu.VMEM((G, SA, EMB_DIM), jnp.float32),
        pltpu.VMEM((G, SB, EMB_DIM), jnp.float32),
        pltpu.VMEM((G, SA, EMB_DIM), jnp.float32),
        pltpu.VMEM((G, SB, EMB_DIM), jnp.float32),
        pltpu.VMEM((BPW, EMB_DIM), jnp.float32),
        pltpu.SemaphoreType.DMA,
        pltpu.SemaphoreType.DMA,
        pltpu.SemaphoreType.DMA,
        pltpu.SemaphoreType.DMA,
    ],
)(_sc_pool_body)


def _mlp_body(ids_ref, sums_ref, w1_ref, b1_ref, w2_ref, b2_ref, out_ref):
    cnt = jnp.sum((ids_ref[...] != 0).astype(jnp.float32), axis=1,
                  keepdims=True)
    avg = sums_ref[...] / jnp.maximum(cnt, 1e-6)
    h = jnp.dot(avg, w1_ref[...], preferred_element_type=jnp.float32,
                precision=lax.Precision.HIGHEST) + b1_ref[...]
    h = jnp.maximum(h, 0.0)
    out_ref[...] = jnp.dot(h, w2_ref[...], preferred_element_type=jnp.float32,
                           precision=lax.Precision.HIGHEST) + b2_ref[...]


def kernel(ids, emb, W1, b1, W2, b2):
    ids = ids.astype(jnp.int32)
    sums = _sc_pool(ids, emb)

    blk = 512
    grid = (BATCH // blk,)
    hidden = W1.shape[1]
    out_dim = W2.shape[1]
    out = pl.pallas_call(
        _mlp_body,
        grid=grid,
        in_specs=[
            pl.BlockSpec((blk, SEQ), lambda i: (i, 0)),
            pl.BlockSpec((blk, EMB_DIM), lambda i: (i, 0)),
            pl.BlockSpec((EMB_DIM, hidden), lambda i: (0, 0)),
            pl.BlockSpec((1, hidden), lambda i: (0, 0)),
            pl.BlockSpec((hidden, out_dim), lambda i: (0, 0)),
            pl.BlockSpec((1, out_dim), lambda i: (0, 0)),
        ],
        out_specs=pl.BlockSpec((blk, out_dim), lambda i: (i, 0)),
        out_shape=jax.ShapeDtypeStruct((BATCH, out_dim), jnp.float32),
    )(ids, sums, W1, b1[None, :], W2, b2[None, :])
    return out
